# baseline (speedup 1.0000x reference)
import functools

import jax
import jax.numpy as jnp
from jax import lax
from jax.experimental import pallas as pl
from jax.experimental.pallas import tpu as pltpu
from jax.experimental.pallas import tpu_sc as plsc

_H = 384
_W = 384
_MH = 28
_MW = 28
_NW = 32
_LANES = 16


def _tc_zero(n_tasks):
    total = n_tasks * _H * _W
    cols = 1024
    rows = total // cols
    grid = 36
    blk = rows // grid

    def body(o_ref):
        o_ref[...] = jnp.zeros((blk, cols), jnp.float32)

    out = pl.pallas_call(
        body,
        out_shape=jax.ShapeDtypeStruct((rows, cols), jnp.float32),
        grid=(grid,),
        out_specs=pl.BlockSpec((blk, cols), lambda i: (i, 0)),
    )()
    return out.reshape(total)


def _sc_paste_boxes(pi, pf, masks, buf, n_tasks):
    mesh = plsc.VectorSubcoreMesh(core_axis_name="c", subcore_axis_name="s")

    @functools.partial(
        pl.kernel,
        mesh=mesh,
        compiler_params=pltpu.CompilerParams(needs_layout_passes=False),
        scratch_types=[
            pltpu.VMEM((n_tasks * _LANES,), jnp.int32),
            pltpu.VMEM((n_tasks * _LANES,), jnp.float32),
            pltpu.VMEM((2, _MH * _MW), jnp.float32),
            pltpu.VMEM((_MH * 32,), jnp.float32),
            pltpu.VMEM((_W,), jnp.int32),
            pltpu.VMEM((_H,), jnp.int32),
            pltpu.VMEM((2 * _MH * _W,), jnp.float32),
            pltpu.SemaphoreType.DMA((2,)),
            pltpu.SemaphoreType.DMA,
        ],
    )
    def k(pi_hbm, pf_hbm, masks_hbm, out_hbm, pi_v, pf_v, mraw, mpad, cidx_v,
          ridx_v, rxf, sem, sem_m):
        cid = lax.axis_index("c")
        sid = lax.axis_index("s")
        wid = sid * 2 + cid

        pltpu.sync_copy(pi_hbm, pi_v)
        pltpu.sync_copy(pf_hbm, pf_v)

        lane = lax.iota(jnp.int32, _LANES)

        n_mine = jnp.where(wid < n_tasks % _NW, 1, 0) + n_tasks // _NW

        def boxh(tt):
            v = pi_v[pl.ds(pl.multiple_of(tt * _LANES, _LANES), _LANES)]
            return jnp.maximum(v[2] - v[0], 0)

        pltpu.async_copy(masks_hbm.at[wid], mraw.at[0], sem_m)

        def task(i, _):
            t = wid + i * _NW
            ib = i & 1
            bb = ib * (_MH * _W)
            smy = sem.at[ib]

            @pl.when(i >= 2)
            def _():
                def drain(q, _):
                    pltpu.make_async_copy(
                        rxf.at[pl.ds(0, _W)],
                        out_hbm.at[pl.ds(0, _W)], smy).wait()
                    return 0

                lax.fori_loop(0, boxh(t - 2 * _NW), drain, 0)

            pltpu.make_async_copy(masks_hbm.at[t], mraw.at[ib],
                                  sem_m).wait()

            @pl.when(i + 1 < n_mine)
            def _():
                pltpu.async_copy(masks_hbm.at[t + _NW], mraw.at[1 - ib],
                                 sem_m)

            vi = pi_v[pl.ds(pl.multiple_of(t * _LANES, _LANES), _LANES)]
            vf = pf_v[pl.ds(pl.multiple_of(t * _LANES, _LANES), _LANES)]
            y1 = vi[0]
            x1 = vi[1]
            y2 = vi[2]
            x2 = vi[3]
            ry = vf[4]
            rx = vf[5]

            ibv = jnp.full((_LANES,), ib, jnp.int32)

            def pad_row(r, _):
                v0 = plsc.load_gather(mraw, [ibv, r * _MW + lane])
                mpad[pl.ds(r * 32, _LANES)] = v0
                hi = jnp.minimum(r * _MW + _LANES + lane, _MH * _MW - 1)
                v1 = plsc.load_gather(mraw, [ibv, hi])
                v1 = jnp.where(lane < _MW - _LANES, v1, 0.0)
                mpad[pl.ds(r * 32 + _LANES, _LANES)] = v1
                return 0

            lax.fori_loop(0, _MH, pad_row, 0)

            for j in range(_W // _LANES):
                x = lane + j * _LANES
                cx = x - x1
                ci = (cx.astype(jnp.float32) * rx).astype(jnp.int32)
                ci = jnp.minimum(jnp.maximum(ci, 0), _MW - 1)
                ok = (x >= x1) & (x < x2)
                cidx_v[pl.ds(j * _LANES, _LANES)] = jnp.where(ok, ci, _MW)

            for j in range(_H // _LANES):
                y = lane + j * _LANES
                dy = y - y1
                ri = (dy.astype(jnp.float32) * ry).astype(jnp.int32)
                ridx_v[pl.ds(j * _LANES, _LANES)] = jnp.minimum(
                    jnp.maximum(ri, 0), _MH - 1)

            def rx_row(r, _):
                base = r * 32
                for j in range(_W // _LANES):
                    idx = cidx_v[pl.ds(j * _LANES, _LANES)] + base
                    rxf[pl.ds(bb + r * _W + j * _LANES,
                              _LANES)] = plsc.load_gather(mpad, [idx])
                return 0

            lax.fori_loop(0, _MH, rx_row, 0)

            obase = t * (_H * _W)

            def row1(y, _):
                rv = plsc.load_gather(ridx_v,
                                      [jnp.full((_LANES,), y, jnp.int32)])
                src = rv[0]
                pltpu.async_copy(
                    rxf.at[pl.ds(bb + src * _W, _W)],
                    out_hbm.at[pl.ds(obase + y * _W, _W)], smy)
                return 0

            lax.fori_loop(y1, y2, row1, 0)
            return 0

        lax.fori_loop(0, n_mine, task, 0)

        def tail(back):
            il = n_mine - back
            tl = wid + il * _NW

            @pl.when(il >= 0)
            def _():
                def drain(q, _):
                    pltpu.make_async_copy(
                        rxf.at[pl.ds(0, _W)],
                        out_hbm.at[pl.ds(0, _W)], sem.at[il & 1]).wait()
                    return 0

                lax.fori_loop(0, boxh(tl), drain, 0)

        tail(2)
        tail(1)

    k(pi, pf, masks, buf)


def kernel(bboxess, counts, maskss, img_h, img_w):
    B, K = maskss.shape[0], maskss.shape[1]
    n = B * K

    boxes = jnp.round(bboxess).astype(jnp.int32)
    y1 = jnp.clip(boxes[..., 0], 0, img_h - 1)
    x1 = jnp.clip(boxes[..., 1], 0, img_w - 1)
    y2 = jnp.clip(boxes[..., 2], y1 + 1, img_h)
    x2 = jnp.clip(boxes[..., 3], x1 + 1, img_w)
    active = jnp.arange(K, dtype=jnp.int32)[None, :] < counts
    y2 = jnp.where(active, y2, y1)
    ratio_y = _MH / jnp.maximum(y2 - y1, 1).astype(jnp.float32)
    ratio_x = _MW / (x2 - x1).astype(jnp.float32)

    zi = jnp.zeros_like(y1)
    pi = jnp.stack([y1, x1, y2, x2] + [zi] * 12, axis=-1)
    pi = pi.reshape(n * _LANES).astype(jnp.int32)
    zf = jnp.zeros_like(ratio_y)
    pf = jnp.stack([zf, zf, zf, zf, ratio_y, ratio_x] + [zf] * 10, axis=-1)
    pf = pf.reshape(n * _LANES).astype(jnp.float32)
    masks = maskss.reshape(n, _MH * _MW).astype(jnp.float32)

    buf = jax.new_ref(_tc_zero(n))
    _sc_paste_boxes(pi, pf, masks, buf, n)
    out = buf[...]
    return out.reshape(B, K, 1, _H, _W)


# speedup vs baseline: 1.4730x; 1.4730x over previous
import functools

import jax
import jax.numpy as jnp
from jax import lax
from jax.experimental import pallas as pl
from jax.experimental.pallas import tpu as pltpu
from jax.experimental.pallas import tpu_sc as plsc

_H = 384
_W = 384
_MH = 28
_MW = 28
_NW = 32
_LANES = 16
_ZROW = _MH
_ZOFF = 2 * _MH * _W
_ZCHUNK = 16


def _sc_paste(pi, pf, masks, n_tasks):
    mesh = plsc.VectorSubcoreMesh(core_axis_name="c", subcore_axis_name="s")

    @functools.partial(
        pl.kernel,
        out_type=jax.ShapeDtypeStruct((n_tasks * _H * _W,), jnp.float32),
        mesh=mesh,
        compiler_params=pltpu.CompilerParams(needs_layout_passes=False),
        scratch_types=[
            pltpu.VMEM((n_tasks * _LANES,), jnp.int32),
            pltpu.VMEM((n_tasks * _LANES,), jnp.float32),
            pltpu.VMEM((2, _MH * _MW), jnp.float32),
            pltpu.VMEM((_MH * 32,), jnp.float32),
            pltpu.VMEM((_W,), jnp.int32),
            pltpu.VMEM((_H,), jnp.int32),
            pltpu.VMEM(((2 * _MH + _ZCHUNK) * _W,), jnp.float32),
            pltpu.SemaphoreType.DMA((2,)),
            pltpu.SemaphoreType.DMA,
        ],
    )
    def k(pi_hbm, pf_hbm, masks_hbm, out_hbm, pi_v, pf_v, mraw, mpad, cidx_v,
          ridx_v, rxf, sem, sem_m):
        cid = lax.axis_index("c")
        sid = lax.axis_index("s")
        wid = sid * 2 + cid

        pltpu.sync_copy(pi_hbm, pi_v)
        pltpu.sync_copy(pf_hbm, pf_v)

        lane = lax.iota(jnp.int32, _LANES)
        zeros16 = jnp.zeros((_LANES,), jnp.float32)

        def zinit(q, _):
            rxf[pl.ds(_ZOFF + q * _LANES, _LANES)] = zeros16
            return 0

        lax.fori_loop(0, _ZCHUNK * _W // _LANES, zinit, 0)

        n_mine = jnp.where(wid < n_tasks % _NW, 1, 0) + n_tasks // _NW

        pltpu.async_copy(masks_hbm.at[wid], mraw.at[0], sem_m)

        def task(i, _):
            t = wid + i * _NW
            ib = i & 1
            bb = ib * (_MH * _W)
            smy = sem.at[ib]

            @pl.when(i >= 2)
            def _():
                def drain(q, _):
                    pltpu.make_async_copy(
                        rxf.at[pl.ds(0, 24 * _W)],
                        out_hbm.at[pl.ds(0, 24 * _W)], smy).wait()
                    return 0

                lax.fori_loop(0, _H // 24, drain, 0)

            pltpu.make_async_copy(masks_hbm.at[t], mraw.at[ib],
                                  sem_m).wait()

            @pl.when(i + 1 < n_mine)
            def _():
                pltpu.async_copy(masks_hbm.at[t + _NW], mraw.at[1 - ib],
                                 sem_m)

            vi = pi_v[pl.ds(pl.multiple_of(t * _LANES, _LANES), _LANES)]
            vf = pf_v[pl.ds(pl.multiple_of(t * _LANES, _LANES), _LANES)]
            y1 = vi[0]
            x1 = vi[1]
            y2 = vi[2]
            x2 = vi[3]
            ry = vf[4]
            rx = vf[5]

            ibv = jnp.full((_LANES,), ib, jnp.int32)

            def pad_row(r, _):
                v0 = plsc.load_gather(mraw, [ibv, r * _MW + lane])
                mpad[pl.ds(r * 32, _LANES)] = v0
                hi = jnp.minimum(r * _MW + _LANES + lane, _MH * _MW - 1)
                v1 = plsc.load_gather(mraw, [ibv, hi])
                v1 = jnp.where(lane < _MW - _LANES, v1, 0.0)
                mpad[pl.ds(r * 32 + _LANES, _LANES)] = v1
                return 0

            lax.fori_loop(0, _MH, pad_row, 0)

            for j in range(_W // _LANES):
                x = lane + j * _LANES
                cx = x - x1
                ci = (cx.astype(jnp.float32) * rx).astype(jnp.int32)
                ci = jnp.minimum(jnp.maximum(ci, 0), _MW - 1)
                ok = (x >= x1) & (x < x2)
                cidx_v[pl.ds(j * _LANES, _LANES)] = jnp.where(ok, ci, _MW)

            for j in range(_H // _LANES):
                y = lane + j * _LANES
                dy = y - y1
                ri = (dy.astype(jnp.float32) * ry).astype(jnp.int32)
                ri = jnp.minimum(jnp.maximum(ri, 0), _MH - 1)
                ok = (y >= y1) & (y < y2)
                ridx_v[pl.ds(j * _LANES, _LANES)] = jnp.where(ok, ri, _ZROW)

            def rx_row(r, _):
                base = r * 32
                for j in range(_W // _LANES):
                    idx = cidx_v[pl.ds(j * _LANES, _LANES)] + base
                    rxf[pl.ds(bb + r * _W + j * _LANES,
                              _LANES)] = plsc.load_gather(mpad, [idx])
                return 0

            lax.fori_loop(0, _MH, rx_row, 0)

            obase = t * (_H * _W)
            nztop = y1 // _ZCHUNK

            def ztop(q, _):
                pltpu.async_copy(
                    rxf.at[pl.ds(_ZOFF, _ZCHUNK * _W)],
                    out_hbm.at[pl.ds(obase + q * (_ZCHUNK * _W),
                                     _ZCHUNK * _W)], smy)
                return 0

            lax.fori_loop(0, nztop, ztop, 0)

            def row1(y, _):
                rv = plsc.load_gather(ridx_v,
                                      [jnp.full((_LANES,), y, jnp.int32)])
                src = rv[0]
                soff = jnp.where(src == _ZROW, _ZOFF, bb + src * _W)
                pltpu.async_copy(
                    rxf.at[pl.ds(soff, _W)],
                    out_hbm.at[pl.ds(obase + y * _W, _W)], smy)
                return 0

            lax.fori_loop(nztop * _ZCHUNK, y2, row1, 0)

            nzbot = (_H - y2) // _ZCHUNK
            y2r = _H - nzbot * _ZCHUNK
            lax.fori_loop(y2, y2r, row1, 0)

            def zbot(q, _):
                pltpu.async_copy(
                    rxf.at[pl.ds(_ZOFF, _ZCHUNK * _W)],
                    out_hbm.at[pl.ds(obase + (y2r + q * _ZCHUNK) * _W,
                                     _ZCHUNK * _W)], smy)
                return 0

            lax.fori_loop(0, nzbot, zbot, 0)
            return 0

        lax.fori_loop(0, n_mine, task, 0)

        def drain_tail(q, _):
            pltpu.make_async_copy(rxf.at[pl.ds(0, 24 * _W)],
                                  out_hbm.at[pl.ds(0, 24 * _W)],
                                  sem.at[0]).wait()
            pltpu.make_async_copy(rxf.at[pl.ds(0, 24 * _W)],
                                  out_hbm.at[pl.ds(0, 24 * _W)],
                                  sem.at[1]).wait()
            return 0

        lax.fori_loop(0, _H // 24, drain_tail, 0)

    return k(pi, pf, masks)


def kernel(bboxess, counts, maskss, img_h, img_w):
    B, K = maskss.shape[0], maskss.shape[1]
    n = B * K

    boxes = jnp.round(bboxess).astype(jnp.int32)
    y1 = jnp.clip(boxes[..., 0], 0, img_h - 1)
    x1 = jnp.clip(boxes[..., 1], 0, img_w - 1)
    y2 = jnp.clip(boxes[..., 2], y1 + 1, img_h)
    x2 = jnp.clip(boxes[..., 3], x1 + 1, img_w)
    active = jnp.arange(K, dtype=jnp.int32)[None, :] < counts
    y2 = jnp.where(active, y2, y1)
    ratio_y = _MH / jnp.maximum(y2 - y1, 1).astype(jnp.float32)
    ratio_x = _MW / (x2 - x1).astype(jnp.float32)

    zi = jnp.zeros_like(y1)
    pi = jnp.stack([y1, x1, y2, x2] + [zi] * 12, axis=-1)
    pi = pi.reshape(n * _LANES).astype(jnp.int32)
    zf = jnp.zeros_like(ratio_y)
    pf = jnp.stack([zf, zf, zf, zf, ratio_y, ratio_x] + [zf] * 10, axis=-1)
    pf = pf.reshape(n * _LANES).astype(jnp.float32)
    masks = maskss.reshape(n, _MH * _MW).astype(jnp.float32)

    out = _sc_paste(pi, pf, masks, n)
    return out.reshape(B, K, 1, _H, _W)
